# baseline (speedup 1.0000x reference)
import functools

import jax
import jax.numpy as jnp
from jax import lax
from jax.experimental import pallas as pl
from jax.experimental.pallas import tpu as pltpu
from jax.experimental.pallas import tpu_sc as plsc

N = 10000
NPAD = 10240
PADNODE = 10016
E = 320000
NC = 2
NS = 16
NW = NC * NS
C = 128
CPW = 80
EPW = CPW * C
EPAD = EPW * NW
ROWS_PER_SUB = NPAD // NS

_mesh = plsc.VectorSubcoreMesh(core_axis_name="c", subcore_axis_name="s")


@functools.partial(
    pl.kernel,
    mesh=_mesh,
    out_type=jax.ShapeDtypeStruct((NC, NPAD, 128), jnp.float32),
    scratch_types=[
        pltpu.VMEM((CPW, C), jnp.int32),
        pltpu.VMEM((CPW, C), jnp.int32),
        pltpu.VMEM((C, 128), jnp.float32),
        pltpu.VMEM_SHARED((NPAD, 128), jnp.float32),
        pltpu.SemaphoreType.DMA,
    ],
)
def _edge_scatter(g_hbm, src_hbm, dst_hbm, zeros_hbm, out_hbm,
                  sv, dv, rows, acc, sem):
  cid = lax.axis_index("c")
  sid = lax.axis_index("s")
  wid = sid * NC + cid
  rbase = sid * ROWS_PER_SUB
  pltpu.sync_copy(src_hbm.at[pl.ds(wid * CPW, CPW)], sv)
  pltpu.sync_copy(dst_hbm.at[pl.ds(wid * CPW, CPW)], dv)
  pltpu.sync_copy(zeros_hbm.at[pl.ds(rbase, ROWS_PER_SUB)],
                  acc.at[pl.ds(rbase, ROWS_PER_SUB)])
  plsc.subcore_barrier()

  def body(t, carry):
    pltpu.async_copy(g_hbm.at[sv.at[t]], rows, sem).wait()
    pltpu.sync_copy(rows, acc.at[dv.at[t]], add=True)
    return carry

  lax.fori_loop(0, CPW, body, 0)
  plsc.subcore_barrier()
  pltpu.sync_copy(acc.at[pl.ds(rbase, ROWS_PER_SUB)],
                  out_hbm.at[cid].at[pl.ds(rbase, ROWS_PER_SUB)])


DEGW = 128


@functools.partial(
    pl.kernel,
    mesh=_mesh,
    out_type=jax.ShapeDtypeStruct((NC, NPAD, DEGW), jnp.float32),
    scratch_types=[
        pltpu.VMEM((CPW, C), jnp.int32),
        pltpu.VMEM((C, DEGW), jnp.float32),
        pltpu.VMEM_SHARED((NPAD, DEGW), jnp.float32),
    ],
)
def _deg_kernel(dst_hbm, zeros_hbm, ones_hbm, out_hbm, dv, ones_v, acc):
  cid = lax.axis_index("c")
  sid = lax.axis_index("s")
  wid = sid * NC + cid
  base = sid * ROWS_PER_SUB
  pltpu.sync_copy(dst_hbm.at[pl.ds(wid * CPW, CPW)], dv)
  pltpu.sync_copy(ones_hbm, ones_v)
  pltpu.sync_copy(zeros_hbm.at[pl.ds(base, ROWS_PER_SUB)],
                  acc.at[pl.ds(base, ROWS_PER_SUB)])
  plsc.subcore_barrier()

  def body(t, carry):
    pltpu.sync_copy(ones_v, acc.at[dv.at[t]], add=True)
    return carry

  lax.fori_loop(0, CPW, body, 0)
  plsc.subcore_barrier()
  pltpu.sync_copy(acc.at[pl.ds(base, ROWS_PER_SUB)],
                  out_hbm.at[cid].at[pl.ds(base, ROWS_PER_SUB)])


R = 1024
GRID = NPAD // R


def _dinv_body(degp_ref, o_ref):
  deg = degp_ref[0, :, 0] + degp_ref[1, :, 0] + 1.0
  o_ref[...] = lax.rsqrt(deg)[:, None]


def _mm1_body(x_ref, w_ref, dinv_ref, o_ref):
  h = jnp.dot(x_ref[...], w_ref[...], preferred_element_type=jnp.float32)
  o_ref[...] = h * dinv_ref[...]


def _mm2_body(s_ref, g1_ref, dinv_ref, b1_ref, w2_ref, o_ref):
  dinv = dinv_ref[...]
  a = (s_ref[0] + s_ref[1] + g1_ref[...]) * dinv + b1_ref[...]
  a = jnp.maximum(a, 0.0)
  h = jnp.dot(a, w2_ref[...], preferred_element_type=jnp.float32)
  o_ref[...] = jnp.concatenate(
      [h * dinv, jnp.zeros((R, 64), jnp.float32)], axis=1)


def _fin_body(s_ref, g2_ref, dinv_ref, b2_ref, o_ref):
  z = ((s_ref[0, :, :64] + s_ref[1, :, :64] + g2_ref[:, :64])
       * dinv_ref[...] + b2_ref[...])
  m = jnp.max(z, axis=1, keepdims=True)
  zs = z - m
  o_ref[...] = zs - jnp.log(jnp.sum(jnp.exp(zs), axis=1, keepdims=True))


def _row_spec(width):
  return pl.BlockSpec((R, width), lambda i: (i, 0))


def _pair_spec(width):
  return pl.BlockSpec((NC, R, width), lambda i: (0, i, 0))


_dinv_spec = pl.BlockSpec((R, 1), lambda i: (i, 0))
_full = lambda shape: pl.BlockSpec(shape, lambda i: (0,) * len(shape))


def _dinv_call(degp):
  return pl.pallas_call(
      _dinv_body,
      grid=(1,),
      in_specs=[pl.BlockSpec((NC, NPAD, DEGW), lambda i: (0, 0, 0))],
      out_specs=pl.BlockSpec((NPAD, 1), lambda i: (0, 0)),
      out_shape=jax.ShapeDtypeStruct((NPAD, 1), jnp.float32),
  )(degp)


def _mm1_call(x, W1, dinv):
  return pl.pallas_call(
      _mm1_body,
      grid=(GRID,),
      in_specs=[_row_spec(128), _full((128, 128)), _dinv_spec],
      out_specs=_row_spec(128),
      out_shape=jax.ShapeDtypeStruct((NPAD, 128), jnp.float32),
  )(x, W1, dinv)


def _mm2_call(s1, g1, dinv, b1, W2):
  return pl.pallas_call(
      _mm2_body,
      grid=(GRID,),
      in_specs=[_pair_spec(128), _row_spec(128), _dinv_spec,
                _full((1, 128)), _full((128, 64))],
      out_specs=_row_spec(128),
      out_shape=jax.ShapeDtypeStruct((NPAD, 128), jnp.float32),
  )(s1, g1, dinv, b1, W2)


def _fin_call(s2, g2, dinv, b2):
  return pl.pallas_call(
      _fin_body,
      grid=(GRID,),
      in_specs=[_pair_spec(128), _row_spec(128), _dinv_spec, _full((1, 64))],
      out_specs=_row_spec(64),
      out_shape=jax.ShapeDtypeStruct((N, 64), jnp.float32),
  )(s2, g2, dinv, b2)


@jax.jit
def kernel(x, edge_index, W1, b1, W2, b2):
  ei = edge_index.astype(jnp.int32)
  pad_src = jnp.full((EPAD - E,), PADNODE, jnp.int32)
  pad_dst = N + jnp.arange(EPAD - E, dtype=jnp.int32) % (NPAD - N)
  src = jnp.concatenate([ei[0], pad_src]).reshape(EPAD // C, C)
  dst = jnp.concatenate([ei[1], pad_dst]).reshape(EPAD // C, C)
  xp = jnp.pad(x, ((0, NPAD - N), (0, 0)))
  zeros128 = jnp.zeros((NPAD, 128), jnp.float32)
  ones = jnp.ones((C, DEGW), jnp.float32)

  degp = _deg_kernel(dst, zeros128, ones)
  dinv = _dinv_call(degp)
  g1 = _mm1_call(xp, W1, dinv)
  s1 = _edge_scatter(g1, src, dst, zeros128)
  g2 = _mm2_call(s1, g1, dinv, b1.reshape(1, -1), W2)
  s2 = _edge_scatter(g2, src, dst, zeros128)
  return _fin_call(s2, g2, dinv, b2.reshape(1, -1))


# speedup vs baseline: 2.2519x; 2.2519x over previous
import functools

import jax
import jax.numpy as jnp
from jax import lax
from jax.experimental import pallas as pl
from jax.experimental.pallas import tpu as pltpu
from jax.experimental.pallas import tpu_sc as plsc

N = 10000
NPAD = 10240
PADNODE = 10016
E = 320000
NC = 2
NS = 16
NW = NC * NS
C = 128
CPW = 80
EPW = CPW * C
EPAD = EPW * NW
ROWS_PER_SUB = NPAD // NS

_mesh = plsc.VectorSubcoreMesh(core_axis_name="c", subcore_axis_name="s")


@functools.partial(
    pl.kernel,
    mesh=_mesh,
    out_type=jax.ShapeDtypeStruct((NC, NPAD, 128), jnp.float32),
    scratch_types=[
        pltpu.VMEM((CPW, C), jnp.int32),
        pltpu.VMEM((CPW, C), jnp.int32),
        pltpu.VMEM((C, 128), jnp.float32),
        pltpu.VMEM_SHARED((NPAD, 128), jnp.float32),
        pltpu.SemaphoreType.DMA,
    ],
)
def _edge_scatter(g_hbm, src_hbm, dst_hbm, zeros_hbm, out_hbm,
                  sv, dv, rows, acc, sem):
  cid = lax.axis_index("c")
  sid = lax.axis_index("s")
  wid = sid * NC + cid
  rbase = sid * ROWS_PER_SUB
  pltpu.sync_copy(src_hbm.at[pl.ds(wid * CPW, CPW)], sv)
  pltpu.sync_copy(dst_hbm.at[pl.ds(wid * CPW, CPW)], dv)
  pltpu.sync_copy(zeros_hbm.at[pl.ds(rbase, ROWS_PER_SUB)],
                  acc.at[pl.ds(rbase, ROWS_PER_SUB)])
  plsc.subcore_barrier()

  def body(t, carry):
    pltpu.async_copy(g_hbm.at[sv.at[t]], rows, sem).wait()
    pltpu.sync_copy(rows, acc.at[dv.at[t]], add=True)
    return carry

  lax.fori_loop(0, CPW, body, 0)
  plsc.subcore_barrier()
  pltpu.sync_copy(acc.at[pl.ds(rbase, ROWS_PER_SUB)],
                  out_hbm.at[cid].at[pl.ds(rbase, ROWS_PER_SUB)])


DEGW = 128


@functools.partial(
    pl.kernel,
    mesh=_mesh,
    out_type=jax.ShapeDtypeStruct((NC, NPAD, DEGW), jnp.float32),
    scratch_types=[
        pltpu.VMEM((CPW, C), jnp.int32),
        pltpu.VMEM((C, DEGW), jnp.float32),
        pltpu.VMEM_SHARED((NPAD, DEGW), jnp.float32),
    ],
)
def _deg_kernel(dst_hbm, zeros_hbm, ones_hbm, out_hbm, dv, ones_v, acc):
  cid = lax.axis_index("c")
  sid = lax.axis_index("s")
  wid = sid * NC + cid
  base = sid * ROWS_PER_SUB
  pltpu.sync_copy(dst_hbm.at[pl.ds(wid * CPW, CPW)], dv)
  pltpu.sync_copy(ones_hbm, ones_v)
  pltpu.sync_copy(zeros_hbm.at[pl.ds(base, ROWS_PER_SUB)],
                  acc.at[pl.ds(base, ROWS_PER_SUB)])
  plsc.subcore_barrier()

  def body(t, carry):
    pltpu.sync_copy(ones_v, acc.at[dv.at[t]], add=True)
    return carry

  lax.fori_loop(0, CPW, body, 0)
  plsc.subcore_barrier()
  pltpu.sync_copy(acc.at[pl.ds(base, ROWS_PER_SUB)],
                  out_hbm.at[cid].at[pl.ds(base, ROWS_PER_SUB)])


R = 1024
GRID = NPAD // R


def _dinv_body(degp_ref, o_ref):
  deg = degp_ref[0, :, 0] + degp_ref[1, :, 0] + 1.0
  o_ref[...] = lax.rsqrt(deg)[:, None]


def _mm1_body(x_ref, w_ref, dinv_ref, o_ref):
  h = jnp.dot(x_ref[...], w_ref[...], preferred_element_type=jnp.float32)
  o_ref[...] = h * dinv_ref[...]


def _mm2_body(s_ref, g1_ref, dinv_ref, b1_ref, w2_ref, o_ref):
  dinv = dinv_ref[...]
  a = (s_ref[0] + s_ref[1] + g1_ref[...]) * dinv + b1_ref[...]
  a = jnp.maximum(a, 0.0)
  h = jnp.dot(a, w2_ref[...], preferred_element_type=jnp.float32)
  o_ref[...] = jnp.concatenate(
      [h * dinv, jnp.zeros((R, 64), jnp.float32)], axis=1)


def _fin_body(s_ref, g2_ref, dinv_ref, b2_ref, o_ref):
  z = ((s_ref[0, :, :64] + s_ref[1, :, :64] + g2_ref[:, :64])
       * dinv_ref[...] + b2_ref[...])
  m = jnp.max(z, axis=1, keepdims=True)
  zs = z - m
  o_ref[...] = zs - jnp.log(jnp.sum(jnp.exp(zs), axis=1, keepdims=True))


def _row_spec(width):
  return pl.BlockSpec((R, width), lambda i: (i, 0))


def _pair_spec(width):
  return pl.BlockSpec((NC, R, width), lambda i: (0, i, 0))


_dinv_spec = pl.BlockSpec((R, 1), lambda i: (i, 0))
_full = lambda shape: pl.BlockSpec(shape, lambda i: (0,) * len(shape))


def _dinv_call(degp):
  return pl.pallas_call(
      _dinv_body,
      grid=(1,),
      in_specs=[pl.BlockSpec((NC, NPAD, DEGW), lambda i: (0, 0, 0))],
      out_specs=pl.BlockSpec((NPAD, 1), lambda i: (0, 0)),
      out_shape=jax.ShapeDtypeStruct((NPAD, 1), jnp.float32),
  )(degp)


def _mm1_call(x, W1, dinv):
  return pl.pallas_call(
      _mm1_body,
      grid=(GRID,),
      in_specs=[_row_spec(128), _full((128, 128)), _dinv_spec],
      out_specs=_row_spec(128),
      out_shape=jax.ShapeDtypeStruct((NPAD, 128), jnp.float32),
  )(x, W1, dinv)


def _mm2_call(s1, g1, dinv, b1, W2):
  return pl.pallas_call(
      _mm2_body,
      grid=(GRID,),
      in_specs=[_pair_spec(128), _row_spec(128), _dinv_spec,
                _full((1, 128)), _full((128, 64))],
      out_specs=_row_spec(128),
      out_shape=jax.ShapeDtypeStruct((NPAD, 128), jnp.float32),
  )(s1, g1, dinv, b1, W2)


def _fin_call(s2, g2, dinv, b2):
  return pl.pallas_call(
      _fin_body,
      grid=(GRID,),
      in_specs=[_pair_spec(128), _row_spec(128), _dinv_spec, _full((1, 64))],
      out_specs=_row_spec(64),
      out_shape=jax.ShapeDtypeStruct((N, 64), jnp.float32),
  )(s2, g2, dinv, b2)


@jax.jit
def kernel(x, edge_index, W1, b1, W2, b2):
  ei = edge_index.astype(jnp.int32)
  pad_iota = N + jnp.arange(EPAD - E, dtype=jnp.int32) % (NPAD - N)
  pad_src = pad_iota
  pad_dst = pad_iota
  src = jnp.concatenate([ei[0], pad_src]).reshape(EPAD // C, C)
  dst = jnp.concatenate([ei[1], pad_dst]).reshape(EPAD // C, C)
  xp = jnp.pad(x, ((0, NPAD - N), (0, 0)))
  zeros128 = jnp.zeros((NPAD, 128), jnp.float32)
  ones = jnp.ones((C, DEGW), jnp.float32)

  degp = _deg_kernel(dst, zeros128, ones)
  dinv = _dinv_call(degp)
  g1 = _mm1_call(xp, W1, dinv)
  s1 = _edge_scatter(g1, src, dst, zeros128)
  g2 = _mm2_call(s1, g1, dinv, b1.reshape(1, -1), W2)
  s2 = _edge_scatter(g2, src, dst, zeros128)
  return _fin_call(s2, g2, dinv, b2.reshape(1, -1))


# speedup vs baseline: 2.4607x; 1.0927x over previous
import functools

import jax
import jax.numpy as jnp
from jax import lax
from jax.experimental import pallas as pl
from jax.experimental.pallas import tpu as pltpu
from jax.experimental.pallas import tpu_sc as plsc

N = 10000
NPAD = 10240
PADNODE = 10016
E = 320000
NC = 2
NS = 16
NW = NC * NS
C = 128
CPW = 80
EPW = CPW * C
EPAD = EPW * NW
ROWS_PER_SUB = NPAD // NS

_mesh = plsc.VectorSubcoreMesh(core_axis_name="c", subcore_axis_name="s")


KB = 2
CPH = CPW // 2


@functools.partial(
    pl.kernel,
    mesh=_mesh,
    out_type=jax.ShapeDtypeStruct((NC, NPAD, 128), jnp.float32),
    scratch_types=[
        pltpu.VMEM((CPH, C), jnp.int32),
        pltpu.VMEM((CPH, C), jnp.int32),
        pltpu.VMEM((C, 128), jnp.float32),
        pltpu.VMEM((C, 128), jnp.float32),
        pltpu.VMEM_SHARED((NPAD, 128), jnp.float32),
        pltpu.SemaphoreType.DMA,
    ],
)
def _edge_scatter(g_hbm, src_hbm, dst_hbm, zeros_hbm, out_hbm,
                  sv, dv, r0, r1, acc, sem):
  cid = lax.axis_index("c")
  sid = lax.axis_index("s")
  wid = sid * NC + cid
  rbase = sid * ROWS_PER_SUB
  rows = (r0, r1)
  pltpu.sync_copy(zeros_hbm.at[pl.ds(rbase, ROWS_PER_SUB)],
                  acc.at[pl.ds(rbase, ROWS_PER_SUB)])
  plsc.subcore_barrier()

  def body(grp, carry):
    for b in range(KB):
      pltpu.async_copy(g_hbm.at[sv.at[grp * KB + b]], rows[b], sem)
    for b in range(KB):
      pltpu.make_async_copy(g_hbm.at[sv.at[grp * KB + b]], rows[b], sem).wait()
    for b in range(KB):
      pltpu.sync_copy(rows[b], acc.at[dv.at[grp * KB + b]], add=True)
    return carry

  for h in range(2):
    pltpu.sync_copy(src_hbm.at[pl.ds(wid * CPW + h * CPH, CPH)], sv)
    pltpu.sync_copy(dst_hbm.at[pl.ds(wid * CPW + h * CPH, CPH)], dv)
    lax.fori_loop(0, CPH // KB, body, 0)

  plsc.subcore_barrier()
  pltpu.sync_copy(acc.at[pl.ds(rbase, ROWS_PER_SUB)],
                  out_hbm.at[cid].at[pl.ds(rbase, ROWS_PER_SUB)])


DEGW = 128


@functools.partial(
    pl.kernel,
    mesh=_mesh,
    out_type=jax.ShapeDtypeStruct((NC, NPAD, DEGW), jnp.float32),
    scratch_types=[
        pltpu.VMEM((CPW, C), jnp.int32),
        pltpu.VMEM((C, DEGW), jnp.float32),
        pltpu.VMEM_SHARED((NPAD, DEGW), jnp.float32),
    ],
)
def _deg_kernel(dst_hbm, zeros_hbm, ones_hbm, out_hbm, dv, ones_v, acc):
  cid = lax.axis_index("c")
  sid = lax.axis_index("s")
  wid = sid * NC + cid
  base = sid * ROWS_PER_SUB
  pltpu.sync_copy(dst_hbm.at[pl.ds(wid * CPW, CPW)], dv)
  pltpu.sync_copy(ones_hbm, ones_v)
  pltpu.sync_copy(zeros_hbm.at[pl.ds(base, ROWS_PER_SUB)],
                  acc.at[pl.ds(base, ROWS_PER_SUB)])
  plsc.subcore_barrier()

  def body(t, carry):
    pltpu.sync_copy(ones_v, acc.at[dv.at[t]], add=True)
    return carry

  lax.fori_loop(0, CPW, body, 0)
  plsc.subcore_barrier()
  pltpu.sync_copy(acc.at[pl.ds(base, ROWS_PER_SUB)],
                  out_hbm.at[cid].at[pl.ds(base, ROWS_PER_SUB)])


R = 1024
GRID = NPAD // R


def _dinv_body(degp_ref, o_ref):
  deg = degp_ref[0, :, 0] + degp_ref[1, :, 0] + 1.0
  o_ref[...] = lax.rsqrt(deg)[:, None]


def _mm1_body(x_ref, w_ref, dinv_ref, o_ref):
  h = jnp.dot(x_ref[...], w_ref[...], preferred_element_type=jnp.float32)
  o_ref[...] = h * dinv_ref[...]


def _mm2_body(s_ref, g1_ref, dinv_ref, b1_ref, w2_ref, o_ref):
  dinv = dinv_ref[...]
  a = (s_ref[0] + s_ref[1] + g1_ref[...]) * dinv + b1_ref[...]
  a = jnp.maximum(a, 0.0)
  h = jnp.dot(a, w2_ref[...], preferred_element_type=jnp.float32)
  o_ref[...] = jnp.concatenate(
      [h * dinv, jnp.zeros((R, 64), jnp.float32)], axis=1)


def _fin_body(s_ref, g2_ref, dinv_ref, b2_ref, o_ref):
  z = ((s_ref[0, :, :64] + s_ref[1, :, :64] + g2_ref[:, :64])
       * dinv_ref[...] + b2_ref[...])
  m = jnp.max(z, axis=1, keepdims=True)
  zs = z - m
  o_ref[...] = zs - jnp.log(jnp.sum(jnp.exp(zs), axis=1, keepdims=True))


def _row_spec(width):
  return pl.BlockSpec((R, width), lambda i: (i, 0))


def _pair_spec(width):
  return pl.BlockSpec((NC, R, width), lambda i: (0, i, 0))


_dinv_spec = pl.BlockSpec((R, 1), lambda i: (i, 0))
_full = lambda shape: pl.BlockSpec(shape, lambda i: (0,) * len(shape))


def _dinv_call(degp):
  return pl.pallas_call(
      _dinv_body,
      grid=(1,),
      in_specs=[pl.BlockSpec((NC, NPAD, DEGW), lambda i: (0, 0, 0))],
      out_specs=pl.BlockSpec((NPAD, 1), lambda i: (0, 0)),
      out_shape=jax.ShapeDtypeStruct((NPAD, 1), jnp.float32),
  )(degp)


def _mm1_call(x, W1, dinv):
  return pl.pallas_call(
      _mm1_body,
      grid=(GRID,),
      in_specs=[_row_spec(128), _full((128, 128)), _dinv_spec],
      out_specs=_row_spec(128),
      out_shape=jax.ShapeDtypeStruct((NPAD, 128), jnp.float32),
  )(x, W1, dinv)


def _mm2_call(s1, g1, dinv, b1, W2):
  return pl.pallas_call(
      _mm2_body,
      grid=(GRID,),
      in_specs=[_pair_spec(128), _row_spec(128), _dinv_spec,
                _full((1, 128)), _full((128, 64))],
      out_specs=_row_spec(128),
      out_shape=jax.ShapeDtypeStruct((NPAD, 128), jnp.float32),
  )(s1, g1, dinv, b1, W2)


def _fin_call(s2, g2, dinv, b2):
  return pl.pallas_call(
      _fin_body,
      grid=(GRID,),
      in_specs=[_pair_spec(128), _row_spec(128), _dinv_spec, _full((1, 64))],
      out_specs=_row_spec(64),
      out_shape=jax.ShapeDtypeStruct((N, 64), jnp.float32),
  )(s2, g2, dinv, b2)


@jax.jit
def kernel(x, edge_index, W1, b1, W2, b2):
  ei = edge_index.astype(jnp.int32)
  pad_iota = N + jnp.arange(EPAD - E, dtype=jnp.int32) % (NPAD - N)
  pad_src = pad_iota
  pad_dst = pad_iota
  src = jnp.concatenate([ei[0], pad_src]).reshape(EPAD // C, C)
  dst = jnp.concatenate([ei[1], pad_dst]).reshape(EPAD // C, C)
  xp = jnp.pad(x, ((0, NPAD - N), (0, 0)))
  zeros128 = jnp.zeros((NPAD, 128), jnp.float32)
  ones = jnp.ones((C, DEGW), jnp.float32)

  degp = _deg_kernel(dst, zeros128, ones)
  dinv = _dinv_call(degp)
  g1 = _mm1_call(xp, W1, dinv)
  s1 = _edge_scatter(g1, src, dst, zeros128)
  g2 = _mm2_call(s1, g1, dinv, b1.reshape(1, -1), W2)
  s2 = _edge_scatter(g2, src, dst, zeros128)
  return _fin_call(s2, g2, dinv, b2.reshape(1, -1))


# speedup vs baseline: 2.4860x; 1.0103x over previous
import functools

import jax
import jax.numpy as jnp
from jax import lax
from jax.experimental import pallas as pl
from jax.experimental.pallas import tpu as pltpu
from jax.experimental.pallas import tpu_sc as plsc

N = 10000
NPAD = 10240
PADNODE = 10016
E = 320000
NC = 2
NS = 16
NW = NC * NS
C = 128
CPW = 80
EPW = CPW * C
EPAD = EPW * NW
ROWS_PER_SUB = NPAD // NS

_mesh = plsc.VectorSubcoreMesh(core_axis_name="c", subcore_axis_name="s")


KB = 2
CPH = CPW // 2


@functools.partial(
    pl.kernel,
    mesh=_mesh,
    out_type=jax.ShapeDtypeStruct((NC, NPAD, 128), jnp.float32),
    scratch_types=[
        pltpu.VMEM((CPH, C), jnp.int32),
        pltpu.VMEM((CPH, C), jnp.int32),
        pltpu.VMEM((C, 128), jnp.float32),
        pltpu.VMEM((C, 128), jnp.float32),
        pltpu.VMEM_SHARED((NPAD, 128), jnp.float32),
        pltpu.SemaphoreType.DMA,
        pltpu.SemaphoreType.DMA,
    ],
)
def _edge_scatter(g_hbm, src_hbm, dst_hbm, zeros_hbm, out_hbm,
                  sv, dv, r0, r1, acc, sem, sadd):
  cid = lax.axis_index("c")
  sid = lax.axis_index("s")
  wid = sid * NC + cid
  rbase = sid * ROWS_PER_SUB
  rows = (r0, r1)
  pltpu.sync_copy(zeros_hbm.at[pl.ds(rbase, ROWS_PER_SUB)],
                  acc.at[pl.ds(rbase, ROWS_PER_SUB)])
  plsc.subcore_barrier()

  def body(grp, carry):
    for b in range(KB):
      pltpu.async_copy(g_hbm.at[sv.at[grp * KB + b]], rows[b], sem)
    for b in range(KB):
      pltpu.make_async_copy(g_hbm.at[sv.at[grp * KB + b]], rows[b], sem).wait()
    for b in range(KB):
      pltpu.async_copy(rows[b], acc.at[dv.at[grp * KB + b]], sadd, add=True)
    for b in range(KB):
      pltpu.make_async_copy(rows[b], acc.at[dv.at[grp * KB + b]], sadd).wait()
    return carry

  for h in range(2):
    pltpu.sync_copy(src_hbm.at[pl.ds(wid * CPW + h * CPH, CPH)], sv)
    pltpu.sync_copy(dst_hbm.at[pl.ds(wid * CPW + h * CPH, CPH)], dv)
    lax.fori_loop(0, CPH // KB, body, 0)

  plsc.subcore_barrier()
  pltpu.sync_copy(acc.at[pl.ds(rbase, ROWS_PER_SUB)],
                  out_hbm.at[cid].at[pl.ds(rbase, ROWS_PER_SUB)])


DEGW = 128


@functools.partial(
    pl.kernel,
    mesh=_mesh,
    out_type=jax.ShapeDtypeStruct((NC, NPAD, DEGW), jnp.float32),
    scratch_types=[
        pltpu.VMEM((CPW, C), jnp.int32),
        pltpu.VMEM((C, DEGW), jnp.float32),
        pltpu.VMEM_SHARED((NPAD, DEGW), jnp.float32),
        pltpu.SemaphoreType.DMA,
    ],
)
def _deg_kernel(dst_hbm, zeros_hbm, ones_hbm, out_hbm, dv, ones_v, acc, sadd):
  cid = lax.axis_index("c")
  sid = lax.axis_index("s")
  wid = sid * NC + cid
  base = sid * ROWS_PER_SUB
  pltpu.sync_copy(dst_hbm.at[pl.ds(wid * CPW, CPW)], dv)
  pltpu.sync_copy(ones_hbm, ones_v)
  pltpu.sync_copy(zeros_hbm.at[pl.ds(base, ROWS_PER_SUB)],
                  acc.at[pl.ds(base, ROWS_PER_SUB)])
  plsc.subcore_barrier()

  def body(grp, carry):
    for b in range(4):
      pltpu.async_copy(ones_v, acc.at[dv.at[grp * 4 + b]], sadd, add=True)
    for b in range(4):
      pltpu.make_async_copy(ones_v, acc.at[dv.at[grp * 4 + b]], sadd).wait()
    return carry

  lax.fori_loop(0, CPW // 4, body, 0)
  plsc.subcore_barrier()
  pltpu.sync_copy(acc.at[pl.ds(base, ROWS_PER_SUB)],
                  out_hbm.at[cid].at[pl.ds(base, ROWS_PER_SUB)])


R = 1024
GRID = NPAD // R


def _dinv_body(degp_ref, o_ref):
  deg = degp_ref[0, :, 0] + degp_ref[1, :, 0] + 1.0
  o_ref[...] = lax.rsqrt(deg)[:, None]


def _mm1_body(x_ref, w_ref, dinv_ref, o_ref):
  h = jnp.dot(x_ref[...], w_ref[...], preferred_element_type=jnp.float32)
  o_ref[...] = h * dinv_ref[...]


def _mm2_body(s_ref, g1_ref, dinv_ref, b1_ref, w2_ref, o_ref):
  dinv = dinv_ref[...]
  a = (s_ref[0] + s_ref[1] + g1_ref[...]) * dinv + b1_ref[...]
  a = jnp.maximum(a, 0.0)
  h = jnp.dot(a, w2_ref[...], preferred_element_type=jnp.float32)
  o_ref[...] = jnp.concatenate(
      [h * dinv, jnp.zeros((R, 64), jnp.float32)], axis=1)


def _fin_body(s_ref, g2_ref, dinv_ref, b2_ref, o_ref):
  z = ((s_ref[0, :, :64] + s_ref[1, :, :64] + g2_ref[:, :64])
       * dinv_ref[...] + b2_ref[...])
  m = jnp.max(z, axis=1, keepdims=True)
  zs = z - m
  o_ref[...] = zs - jnp.log(jnp.sum(jnp.exp(zs), axis=1, keepdims=True))


def _row_spec(width):
  return pl.BlockSpec((R, width), lambda i: (i, 0))


def _pair_spec(width):
  return pl.BlockSpec((NC, R, width), lambda i: (0, i, 0))


_dinv_spec = pl.BlockSpec((R, 1), lambda i: (i, 0))
_full = lambda shape: pl.BlockSpec(shape, lambda i: (0,) * len(shape))


def _dinv_call(degp):
  return pl.pallas_call(
      _dinv_body,
      grid=(1,),
      in_specs=[pl.BlockSpec((NC, NPAD, DEGW), lambda i: (0, 0, 0))],
      out_specs=pl.BlockSpec((NPAD, 1), lambda i: (0, 0)),
      out_shape=jax.ShapeDtypeStruct((NPAD, 1), jnp.float32),
  )(degp)


def _mm1_call(x, W1, dinv):
  return pl.pallas_call(
      _mm1_body,
      grid=(GRID,),
      in_specs=[_row_spec(128), _full((128, 128)), _dinv_spec],
      out_specs=_row_spec(128),
      out_shape=jax.ShapeDtypeStruct((NPAD, 128), jnp.float32),
  )(x, W1, dinv)


def _mm2_call(s1, g1, dinv, b1, W2):
  return pl.pallas_call(
      _mm2_body,
      grid=(GRID,),
      in_specs=[_pair_spec(128), _row_spec(128), _dinv_spec,
                _full((1, 128)), _full((128, 64))],
      out_specs=_row_spec(128),
      out_shape=jax.ShapeDtypeStruct((NPAD, 128), jnp.float32),
  )(s1, g1, dinv, b1, W2)


def _fin_call(s2, g2, dinv, b2):
  return pl.pallas_call(
      _fin_body,
      grid=(GRID,),
      in_specs=[_pair_spec(128), _row_spec(128), _dinv_spec, _full((1, 64))],
      out_specs=_row_spec(64),
      out_shape=jax.ShapeDtypeStruct((N, 64), jnp.float32),
  )(s2, g2, dinv, b2)


@jax.jit
def kernel(x, edge_index, W1, b1, W2, b2):
  ei = edge_index.astype(jnp.int32)
  pad_iota = N + jnp.arange(EPAD - E, dtype=jnp.int32) % (NPAD - N)
  pad_src = pad_iota
  pad_dst = pad_iota
  src = jnp.concatenate([ei[0], pad_src]).reshape(EPAD // C, C)
  dst = jnp.concatenate([ei[1], pad_dst]).reshape(EPAD // C, C)
  xp = jnp.pad(x, ((0, NPAD - N), (0, 0)))
  zeros128 = jnp.zeros((NPAD, 128), jnp.float32)
  ones = jnp.ones((C, DEGW), jnp.float32)

  degp = _deg_kernel(dst, zeros128, ones)
  dinv = _dinv_call(degp)
  g1 = _mm1_call(xp, W1, dinv)
  s1 = _edge_scatter(g1, src, dst, zeros128)
  g2 = _mm2_call(s1, g1, dinv, b1.reshape(1, -1), W2)
  s2 = _edge_scatter(g2, src, dst, zeros128)
  return _fin_call(s2, g2, dinv, b2.reshape(1, -1))


# speedup vs baseline: 2.5212x; 1.0141x over previous
import functools

import jax
import jax.numpy as jnp
from jax import lax
from jax.experimental import pallas as pl
from jax.experimental.pallas import tpu as pltpu
from jax.experimental.pallas import tpu_sc as plsc

N = 10000
NPAD = 10240
PADNODE = 10016
E = 320000
NC = 2
NS = 16
NW = NC * NS
C = 128
CPW = 80
EPW = CPW * C
EPAD = EPW * NW
ROWS_PER_SUB = NPAD // NS

_mesh = plsc.VectorSubcoreMesh(core_axis_name="c", subcore_axis_name="s")


KB = 2
CPH = CPW // 2


@functools.partial(
    pl.kernel,
    mesh=_mesh,
    out_type=jax.ShapeDtypeStruct((NC, NPAD, 128), jnp.float32),
    scratch_types=[
        pltpu.VMEM((CPH, C), jnp.int32),
        pltpu.VMEM((CPH, C), jnp.int32),
        pltpu.VMEM((C, 128), jnp.float32),
        pltpu.VMEM((C, 128), jnp.float32),
        pltpu.VMEM_SHARED((NPAD, 128), jnp.float32),
        pltpu.SemaphoreType.DMA,
        pltpu.SemaphoreType.DMA,
    ],
)
def _edge_scatter(g_hbm, src_hbm, dst_hbm, zeros_hbm, out_hbm,
                  sv, dv, r0, r1, acc, sem, sadd):
  cid = lax.axis_index("c")
  sid = lax.axis_index("s")
  wid = sid * NC + cid
  rbase = sid * ROWS_PER_SUB
  rows = (r0, r1)
  pltpu.sync_copy(zeros_hbm.at[pl.ds(rbase, ROWS_PER_SUB)],
                  acc.at[pl.ds(rbase, ROWS_PER_SUB)])
  plsc.subcore_barrier()

  def body(grp, carry):
    for b in range(KB):
      pltpu.async_copy(g_hbm.at[sv.at[grp * KB + b]], rows[b], sem)
    for b in range(KB):
      pltpu.make_async_copy(g_hbm.at[sv.at[grp * KB + b]], rows[b], sem).wait()
    for b in range(KB):
      pltpu.async_copy(rows[b], acc.at[dv.at[grp * KB + b]], sadd, add=True)
    for b in range(KB):
      pltpu.make_async_copy(rows[b], acc.at[dv.at[grp * KB + b]], sadd).wait()
    return carry

  for h in range(2):
    pltpu.sync_copy(src_hbm.at[pl.ds(wid * CPW + h * CPH, CPH)], sv)
    pltpu.sync_copy(dst_hbm.at[pl.ds(wid * CPW + h * CPH, CPH)], dv)
    lax.fori_loop(0, CPH // KB, body, 0)

  plsc.subcore_barrier()
  pltpu.sync_copy(acc.at[pl.ds(rbase, ROWS_PER_SUB)],
                  out_hbm.at[cid].at[pl.ds(rbase, ROWS_PER_SUB)])


DEGW = 128


@functools.partial(
    pl.kernel,
    mesh=_mesh,
    out_type=jax.ShapeDtypeStruct((NC, NPAD, DEGW), jnp.float32),
    scratch_types=[
        pltpu.VMEM((CPW, C), jnp.int32),
        pltpu.VMEM((C, DEGW), jnp.float32),
        pltpu.VMEM_SHARED((NPAD, DEGW), jnp.float32),
        pltpu.SemaphoreType.DMA,
    ],
)
def _deg_kernel(dst_hbm, zeros_hbm, ones_hbm, out_hbm, dv, ones_v, acc, sadd):
  cid = lax.axis_index("c")
  sid = lax.axis_index("s")
  wid = sid * NC + cid
  base = sid * ROWS_PER_SUB
  pltpu.sync_copy(dst_hbm.at[pl.ds(wid * CPW, CPW)], dv)
  pltpu.sync_copy(ones_hbm, ones_v)
  pltpu.sync_copy(zeros_hbm.at[pl.ds(base, ROWS_PER_SUB)],
                  acc.at[pl.ds(base, ROWS_PER_SUB)])
  plsc.subcore_barrier()

  def body(grp, carry):
    for b in range(4):
      pltpu.async_copy(ones_v, acc.at[dv.at[grp * 4 + b]], sadd, add=True)
    for b in range(4):
      pltpu.make_async_copy(ones_v, acc.at[dv.at[grp * 4 + b]], sadd).wait()
    return carry

  lax.fori_loop(0, CPW // 4, body, 0)
  plsc.subcore_barrier()
  pltpu.sync_copy(acc.at[pl.ds(base, ROWS_PER_SUB)],
                  out_hbm.at[cid].at[pl.ds(base, ROWS_PER_SUB)])


R = 1024
GRID = NPAD // R


def _dinv_of(degp_ref):
  deg = degp_ref[0, :, 0] + degp_ref[1, :, 0] + 1.0
  return lax.rsqrt(deg)[:, None]


def _mm1_body(x_ref, w_ref, degp_ref, o_ref):
  h = jnp.dot(x_ref[...], w_ref[...], preferred_element_type=jnp.float32)
  o_ref[...] = h * _dinv_of(degp_ref)


def _mm2_body(s_ref, g1_ref, degp_ref, b1_ref, w2_ref, o_ref):
  dinv = _dinv_of(degp_ref)
  a = (s_ref[0] + s_ref[1] + g1_ref[...]) * dinv + b1_ref[...]
  a = jnp.maximum(a, 0.0)
  h = jnp.dot(a, w2_ref[...], preferred_element_type=jnp.float32)
  o_ref[...] = jnp.concatenate(
      [h * dinv, jnp.zeros((R, 64), jnp.float32)], axis=1)


def _fin_body(s_ref, g2_ref, degp_ref, b2_ref, o_ref):
  z = ((s_ref[0, :, :64] + s_ref[1, :, :64] + g2_ref[:, :64])
       * _dinv_of(degp_ref) + b2_ref[...])
  m = jnp.max(z, axis=1, keepdims=True)
  zs = z - m
  o_ref[...] = zs - jnp.log(jnp.sum(jnp.exp(zs), axis=1, keepdims=True))


def _row_spec(width):
  return pl.BlockSpec((R, width), lambda i: (i, 0))


def _pair_spec(width):
  return pl.BlockSpec((NC, R, width), lambda i: (0, i, 0))


_degp_spec = pl.BlockSpec((NC, R, DEGW), lambda i: (0, i, 0))
_full = lambda shape: pl.BlockSpec(shape, lambda i: (0,) * len(shape))


def _mm1_call(x, W1, degp):
  return pl.pallas_call(
      _mm1_body,
      grid=(GRID,),
      in_specs=[_row_spec(128), _full((128, 128)), _degp_spec],
      out_specs=_row_spec(128),
      out_shape=jax.ShapeDtypeStruct((NPAD, 128), jnp.float32),
  )(x, W1, degp)


def _mm2_call(s1, g1, degp, b1, W2):
  return pl.pallas_call(
      _mm2_body,
      grid=(GRID,),
      in_specs=[_pair_spec(128), _row_spec(128), _degp_spec,
                _full((1, 128)), _full((128, 64))],
      out_specs=_row_spec(128),
      out_shape=jax.ShapeDtypeStruct((NPAD, 128), jnp.float32),
  )(s1, g1, degp, b1, W2)


def _fin_call(s2, g2, degp, b2):
  return pl.pallas_call(
      _fin_body,
      grid=(GRID,),
      in_specs=[_pair_spec(128), _row_spec(128), _degp_spec, _full((1, 64))],
      out_specs=_row_spec(64),
      out_shape=jax.ShapeDtypeStruct((N, 64), jnp.float32),
  )(s2, g2, degp, b2)


@jax.jit
def kernel(x, edge_index, W1, b1, W2, b2):
  ei = edge_index.astype(jnp.int32)
  pad_iota = N + jnp.arange(EPAD - E, dtype=jnp.int32) % (NPAD - N)
  pad_src = pad_iota
  pad_dst = pad_iota
  src = jnp.concatenate([ei[0], pad_src]).reshape(EPAD // C, C)
  dst = jnp.concatenate([ei[1], pad_dst]).reshape(EPAD // C, C)
  xp = jnp.pad(x, ((0, NPAD - N), (0, 0)))
  zeros128 = jnp.zeros((NPAD, 128), jnp.float32)
  ones = jnp.ones((C, DEGW), jnp.float32)

  degp = _deg_kernel(dst, zeros128, ones)
  g1 = _mm1_call(xp, W1, degp)
  s1 = _edge_scatter(g1, src, dst, zeros128)
  g2 = _mm2_call(s1, g1, degp, b1.reshape(1, -1), W2)
  s2 = _edge_scatter(g2, src, dst, zeros128)
  return _fin_call(s2, g2, degp, b2.reshape(1, -1))
